# Initial kernel scaffold; baseline (speedup 1.0000x reference)
import jax
import jax.numpy as jnp
from jax.experimental import pallas as pl


def kernel(x, edge_index, W):
    raise NotImplementedError("write your pallas kernel here")


# speedup vs baseline: 3.9131x; 3.9131x over previous
import functools

import jax
import jax.numpy as jnp
from jax import lax
from jax.experimental import pallas as pl
from jax.experimental.pallas import tpu as pltpu
from jax.experimental.pallas import tpu_sc as plsc

N_NODES = 10000
N_EDGES = 320000
D = 128
DH = D // 2

NC = 2
NS = 16
E_PER_TILE = N_EDGES // NS
CHUNK = 80
N_CHUNKS = E_PER_TILE // CHUNK
N_PAD = 10240
ROWS_PER_TILE = N_PAD // NS
ZROWS = 128
CW = 16


def _relu_body(x_ref, o_ref):
    o_ref[0] = jnp.maximum(x_ref[:, 0:DH], 0.0)
    o_ref[1] = jnp.maximum(x_ref[:, DH:D], 0.0)


def _relu_tc(x):
    blk = 2000
    return pl.pallas_call(
        _relu_body,
        grid=(N_NODES // blk,),
        in_specs=[pl.BlockSpec((blk, D), lambda i: (i, 0))],
        out_specs=pl.BlockSpec((NC, blk, DH), lambda i: (0, i, 0)),
        out_shape=jax.ShapeDtypeStruct((NC, N_NODES, DH), jnp.float32),
    )(x)


def _agg_sc_body(relu_hbm, src_hbm, dst_hbm, part_out, cnt_out,
                 acc, cnt, src_v, dst_v, rows_v, ones_v, zf, zc, sem):
    c = lax.axis_index("c")
    s = lax.axis_index("s")

    @pl.loop(0, ZROWS)
    def _(i):
        for j in range(DH // 16):
            zf[i, pl.ds(j * 16, 16)] = jnp.zeros((16,), jnp.float32)
        zc[i, :] = jnp.zeros((16,), jnp.float32)

    @pl.loop(0, CHUNK)
    def _(i):
        ones_v[i, :] = jnp.ones((16,), jnp.float32)

    row0 = s * ROWS_PER_TILE
    for k in range(ROWS_PER_TILE // ZROWS):
        pltpu.sync_copy(zf, acc.at[pl.ds(row0 + k * ZROWS, ZROWS)])
        pltpu.sync_copy(zc, cnt.at[pl.ds(row0 + k * ZROWS, ZROWS)])
    plsc.subcore_barrier()

    ebase = c * N_EDGES + s * E_PER_TILE

    @pl.loop(0, N_CHUNKS)
    def _(j):
        off = ebase + j * CHUNK
        pltpu.sync_copy(src_hbm.at[pl.ds(off, CHUNK)], src_v)
        pltpu.sync_copy(dst_hbm.at[pl.ds(off, CHUNK)], dst_v)
        pltpu.async_copy(relu_hbm.at[src_v], rows_v, sem).wait()
        pltpu.sync_copy(rows_v, acc.at[dst_v], add=True)

        @pl.when(c == 0)
        def _():
            pltpu.sync_copy(ones_v, cnt.at[dst_v], add=True)

    plsc.subcore_barrier()

    for k in range(ROWS_PER_TILE // ZROWS):
        r = row0 + k * ZROWS
        pltpu.sync_copy(acc.at[pl.ds(r, ZROWS)], part_out.at[c, pl.ds(r, ZROWS)])

        @pl.when(c == 0)
        def _():
            pltpu.sync_copy(cnt.at[pl.ds(r, ZROWS)], cnt_out.at[pl.ds(r, ZROWS)])


_agg_sc = functools.partial(
    pl.kernel,
    out_type=(
        jax.ShapeDtypeStruct((NC, N_PAD, DH), jnp.float32),
        jax.ShapeDtypeStruct((N_PAD, CW), jnp.float32),
    ),
    mesh=plsc.VectorSubcoreMesh(core_axis_name="c", subcore_axis_name="s"),
    compiler_params=pltpu.CompilerParams(use_tc_tiling_on_sc=False),
    scratch_types=[
        pltpu.VMEM_SHARED((N_PAD, DH), jnp.float32),
        pltpu.VMEM_SHARED((N_PAD, CW), jnp.float32),
        pltpu.VMEM((CHUNK,), jnp.int32),
        pltpu.VMEM((CHUNK,), jnp.int32),
        pltpu.VMEM((CHUNK, DH), jnp.float32),
        pltpu.VMEM((CHUNK, CW), jnp.float32),
        pltpu.VMEM((ZROWS, DH), jnp.float32),
        pltpu.VMEM((ZROWS, CW), jnp.float32),
        pltpu.SemaphoreType.DMA,
    ],
)(_agg_sc_body)


def _combine_body(part_ref, cnt_ref, x_ref, w_ref, o_ref):
    inv = 1.0 / jnp.maximum(cnt_ref[:, 0:1], 1.0)
    a0 = part_ref[0] * inv + x_ref[:, 0:DH]
    a1 = part_ref[1] * inv + x_ref[:, DH:D]
    t = (jnp.dot(a0, w_ref[0:DH, :], preferred_element_type=jnp.float32)
         + jnp.dot(a1, w_ref[DH:D, :], preferred_element_type=jnp.float32))
    o_ref[...] = jnp.maximum(t, 0.0)


def _combine_tc(part, cnt, x, W):
    blk = 2000
    return pl.pallas_call(
        _combine_body,
        grid=(N_NODES // blk,),
        in_specs=[
            pl.BlockSpec((NC, blk, DH), lambda i: (0, i, 0)),
            pl.BlockSpec((blk, CW), lambda i: (i, 0)),
            pl.BlockSpec((blk, D), lambda i: (i, 0)),
            pl.BlockSpec((D, D), lambda i: (0, 0)),
        ],
        out_specs=pl.BlockSpec((blk, D), lambda i: (i, 0)),
        out_shape=jax.ShapeDtypeStruct((N_NODES, D), jnp.float32),
    )(part, cnt, x, W)


def kernel(x, edge_index, W):
    src = edge_index[0]
    dst = edge_index[1]
    src_cat = jnp.concatenate([src, src + N_NODES])
    dst_cat = jnp.concatenate([dst, dst])
    relu2 = _relu_tc(x).reshape(NC * N_NODES, DH)
    part, cnt = _agg_sc(relu2, src_cat, dst_cat)
    return _combine_tc(part, cnt, x, W)


# speedup vs baseline: 11.0497x; 2.8238x over previous
import functools

import jax
import jax.numpy as jnp
from jax import lax
from jax.experimental import pallas as pl
from jax.experimental.pallas import tpu as pltpu
from jax.experimental.pallas import tpu_sc as plsc

N_NODES = 10000
N_EDGES = 320000
D = 128
DH = D // 2

NC = 2
NS = 16
E_PER_TILE = N_EDGES // NS
CHUNK = 80
N_CHUNKS = E_PER_TILE // CHUNK
NBUF = 4
LAG = 2
IG = 50
N_IG = N_CHUNKS // IG
N_PAD = 10240
ROWS_PER_TILE = N_PAD // NS
ZROWS = 64
CW = 16
CNT_SPLIT = N_CHUNKS // 2


def _relu_body(x_ref, o_ref):
    o_ref[0] = jnp.maximum(x_ref[:, 0:DH], 0.0)
    o_ref[1] = jnp.maximum(x_ref[:, DH:D], 0.0)


def _relu_tc(x):
    blk = 2000
    return pl.pallas_call(
        _relu_body,
        grid=(N_NODES // blk,),
        in_specs=[pl.BlockSpec((blk, D), lambda i: (i, 0))],
        out_specs=pl.BlockSpec((NC, blk, DH), lambda i: (0, i, 0)),
        out_shape=jax.ShapeDtypeStruct((NC, N_NODES, DH), jnp.float32),
    )(x)


def _agg_sc_body(relu_hbm, src_hbm, dst_hbm, part_out, cnt_out,
                 acc, cnt, src_buf, dst_buf, rows, ones_v, zf, zc,
                 g_sems, s_sems, cnt_sem, is_sem, id_sem):
    c = lax.axis_index("c")
    s = lax.axis_index("s")

    @pl.loop(0, ZROWS)
    def _(i):
        for j in range(DH // 16):
            zf[i, pl.ds(j * 16, 16)] = jnp.zeros((16,), jnp.float32)
        zc[i, :] = jnp.zeros((16,), jnp.float32)

    @pl.loop(0, CHUNK)
    def _(i):
        ones_v[i, :] = jnp.ones((16,), jnp.float32)

    row0 = s * ROWS_PER_TILE
    for k in range(ROWS_PER_TILE // ZROWS):
        pltpu.sync_copy(zf, acc.at[pl.ds(row0 + k * ZROWS, ZROWS)])
        pltpu.sync_copy(zc, cnt.at[pl.ds(row0 + k * ZROWS, ZROWS)])
    plsc.subcore_barrier()

    crow0 = c * (N_EDGES // CHUNK) + s * N_CHUNKS

    def start_idx_load(ig, p):
        r = crow0 + ig * IG
        pltpu.async_copy(src_hbm.at[pl.ds(r, IG)], src_buf.at[p], is_sem)
        pltpu.async_copy(dst_hbm.at[pl.ds(r, IG)], dst_buf.at[p], id_sem)

    def wait_idx_load(p):
        pltpu.make_async_copy(src_hbm.at[pl.ds(crow0, IG)], src_buf.at[p],
                              is_sem).wait()
        pltpu.make_async_copy(dst_hbm.at[pl.ds(crow0, IG)], dst_buf.at[p],
                              id_sem).wait()

    def start_gather(p, l, b):
        pltpu.async_copy(relu_hbm.at[src_buf.at[p, l]], rows.at[b],
                         g_sems.at[b])

    def wait_gather(p, l, b):
        pltpu.make_async_copy(relu_hbm.at[src_buf.at[p, l]], rows.at[b],
                              g_sems.at[b]).wait()

    def start_scatter(p, l, b):
        pltpu.async_copy(rows.at[b], acc.at[dst_buf.at[p, l]],
                         s_sems.at[b], add=True)

    def wait_scatter(p, l, b):
        pltpu.make_async_copy(rows.at[b], acc.at[dst_buf.at[p, l]],
                              s_sems.at[b]).wait()

    def fire_cnt(p, l, jj):
        take = ((c == 0) & (jj < CNT_SPLIT)) | ((c == 1) & (jj >= CNT_SPLIT))

        @pl.when(take)
        def _():
            pltpu.async_copy(ones_v, cnt.at[dst_buf.at[p, l]],
                             cnt_sem, add=True)

    def step(ig, p, l, b, bg, do_gather, wait_sc):
        if do_gather:
            if wait_sc:
                wait_scatter(p, l + LAG - NBUF, bg)
            start_gather(p, l + LAG, bg)
        wait_gather(p, l, b)
        start_scatter(p, l, b)
        fire_cnt(p, l, ig * IG + l)

    pltpu.sync_copy(src_hbm.at[pl.ds(crow0, IG)], src_buf.at[0])
    pltpu.sync_copy(dst_hbm.at[pl.ds(crow0, IG)], dst_buf.at[0])

    for ig in range(N_IG):
        p = ig % 2
        if ig + 1 < N_IG:
            start_idx_load(ig + 1, 1 - p)
        for l in range(LAG):
            start_gather(p, l, l % NBUF)
        for l in range(LAG):
            step(ig, p, l, l % NBUF, (l + LAG) % NBUF,
                 do_gather=True, wait_sc=(l + LAG >= NBUF))

        @pl.loop(0, (IG - LAG - NBUF) // NBUF)
        def _(g):
            l0 = LAG + g * NBUF
            for b in range(NBUF):
                l = l0 + b
                step(ig, p, l, (b + LAG) % NBUF, (b + 2 * LAG) % NBUF,
                     do_gather=True, wait_sc=True)

        for l in range(LAG + ((IG - LAG - NBUF) // NBUF) * NBUF, IG):
            step(ig, p, l, l % NBUF, (l + LAG) % NBUF,
                 do_gather=(l + LAG < IG), wait_sc=(l + LAG >= NBUF))

        for l in range(IG - NBUF, IG):
            wait_scatter(p, l, l % NBUF)
        if ig + 1 < N_IG:
            wait_idx_load(1 - p)

    @pl.loop(0, CNT_SPLIT)
    def _(i):
        pltpu.make_async_copy(ones_v, cnt.at[dst_buf.at[0, 0]],
                              cnt_sem).wait()

    plsc.subcore_barrier()

    for k in range(ROWS_PER_TILE // ZROWS):
        r = row0 + k * ZROWS
        pltpu.sync_copy(acc.at[pl.ds(r, ZROWS)], part_out.at[c, pl.ds(r, ZROWS)])
        pltpu.sync_copy(cnt.at[pl.ds(r, ZROWS)], cnt_out.at[c, pl.ds(r, ZROWS)])


_agg_sc = functools.partial(
    pl.kernel,
    out_type=(
        jax.ShapeDtypeStruct((NC, N_PAD, DH), jnp.float32),
        jax.ShapeDtypeStruct((NC, N_PAD, CW), jnp.float32),
    ),
    mesh=plsc.VectorSubcoreMesh(core_axis_name="c", subcore_axis_name="s"),
    compiler_params=pltpu.CompilerParams(use_tc_tiling_on_sc=False),
    scratch_types=[
        pltpu.VMEM_SHARED((N_PAD, DH), jnp.float32),
        pltpu.VMEM_SHARED((N_PAD, CW), jnp.float32),
        pltpu.VMEM((2, IG, CHUNK), jnp.int32),
        pltpu.VMEM((2, IG, CHUNK), jnp.int32),
        pltpu.VMEM((NBUF, CHUNK, DH), jnp.float32),
        pltpu.VMEM((CHUNK, CW), jnp.float32),
        pltpu.VMEM((ZROWS, DH), jnp.float32),
        pltpu.VMEM((ZROWS, CW), jnp.float32),
        pltpu.SemaphoreType.DMA((NBUF,)),
        pltpu.SemaphoreType.DMA((NBUF,)),
        pltpu.SemaphoreType.DMA,
        pltpu.SemaphoreType.DMA,
        pltpu.SemaphoreType.DMA,
    ],
)(_agg_sc_body)


def _combine_body(part_ref, cnt_ref, x_ref, w_ref, o_ref):
    csum = cnt_ref[0, :, 0:1] + cnt_ref[1, :, 0:1]
    inv = 1.0 / jnp.maximum(csum, 1.0)
    a0 = part_ref[0] * inv + x_ref[:, 0:DH]
    a1 = part_ref[1] * inv + x_ref[:, DH:D]
    t = (jnp.dot(a0, w_ref[0:DH, :], preferred_element_type=jnp.float32)
         + jnp.dot(a1, w_ref[DH:D, :], preferred_element_type=jnp.float32))
    o_ref[...] = jnp.maximum(t, 0.0)


def _combine_tc(part, cnt, x, W):
    blk = 2000
    return pl.pallas_call(
        _combine_body,
        grid=(N_NODES // blk,),
        in_specs=[
            pl.BlockSpec((NC, blk, DH), lambda i: (0, i, 0)),
            pl.BlockSpec((NC, blk, CW), lambda i: (0, i, 0)),
            pl.BlockSpec((blk, D), lambda i: (i, 0)),
            pl.BlockSpec((D, D), lambda i: (0, 0)),
        ],
        out_specs=pl.BlockSpec((blk, D), lambda i: (i, 0)),
        out_shape=jax.ShapeDtypeStruct((N_NODES, D), jnp.float32),
    )(part, cnt, x, W)


def kernel(x, edge_index, W):
    src = edge_index[0]
    dst = edge_index[1]
    src_cat = jnp.concatenate([src, src + N_NODES]).reshape(-1, CHUNK)
    dst_cat = jnp.concatenate([dst, dst]).reshape(-1, CHUNK)
    relu2 = _relu_tc(x).reshape(NC * N_NODES, DH)
    part, cnt = _agg_sc(relu2, src_cat, dst_cat)
    return _combine_tc(part, cnt, x, W)


# speedup vs baseline: 12.0044x; 1.0864x over previous
import functools

import jax
import jax.numpy as jnp
from jax import lax
from jax.experimental import pallas as pl
from jax.experimental.pallas import tpu as pltpu
from jax.experimental.pallas import tpu_sc as plsc

N_NODES = 10000
N_EDGES = 320000
D = 128
DH = D // 2

NC = 2
NS = 16
E_PER_TILE = N_EDGES // NS
CHUNK = 80
N_CHUNKS = E_PER_TILE // CHUNK
NBUF = 4
LAG = 2
IG = 50
N_IG = N_CHUNKS // IG
N_PAD = 10240
ROWS_PER_TILE = N_PAD // NS
ZROWS = 64
CW = 16
CNT_SPLIT = N_CHUNKS // 2


def _relu_body(x_ref, o_ref):
    o_ref[0] = jnp.maximum(x_ref[:, 0:DH], 0.0)
    o_ref[1] = jnp.maximum(x_ref[:, DH:D], 0.0)


def _relu_tc(x):
    blk = 2000
    return pl.pallas_call(
        _relu_body,
        grid=(N_NODES // blk,),
        in_specs=[pl.BlockSpec((blk, D), lambda i: (i, 0))],
        out_specs=pl.BlockSpec((NC, blk, DH), lambda i: (0, i, 0)),
        out_shape=jax.ShapeDtypeStruct((NC, N_NODES, DH), jnp.float32),
    )(x)


def _agg_sc_body(relu_hbm, src_hbm, dst_hbm, part_out, cnt_out,
                 acc, cnt, src_buf, dst_buf, rows, ones_v, zf, zc,
                 g_sems, s_sems, cnt_sem, is_sem, id_sem):
    c = lax.axis_index("c")
    s = lax.axis_index("s")

    @pl.loop(0, ZROWS)
    def _(i):
        for j in range(DH // 16):
            zf[i, pl.ds(j * 16, 16)] = jnp.zeros((16,), jnp.float32)
        zc[i, :] = jnp.zeros((16,), jnp.float32)

    @pl.loop(0, CHUNK)
    def _(i):
        ones_v[i, :] = jnp.ones((16,), jnp.float32)

    row0 = s * ROWS_PER_TILE
    for k in range(ROWS_PER_TILE // ZROWS):
        pltpu.sync_copy(zf, acc.at[pl.ds(row0 + k * ZROWS, ZROWS)])
        pltpu.sync_copy(zc, cnt.at[pl.ds(row0 + k * ZROWS, ZROWS)])
    plsc.subcore_barrier()

    crow0 = s * N_CHUNKS

    def start_idx_load(ig, p):
        r = crow0 + ig * IG
        pltpu.async_copy(src_hbm.at[pl.ds(r, IG)], src_buf.at[p], is_sem)
        pltpu.async_copy(dst_hbm.at[pl.ds(r, IG)], dst_buf.at[p], id_sem)

    def wait_idx_load(p):
        pltpu.make_async_copy(src_hbm.at[pl.ds(crow0, IG)], src_buf.at[p],
                              is_sem).wait()
        pltpu.make_async_copy(dst_hbm.at[pl.ds(crow0, IG)], dst_buf.at[p],
                              id_sem).wait()

    def start_gather(p, l, b):
        pltpu.async_copy(relu_hbm.at[c].at[src_buf.at[p, l]], rows.at[b],
                         g_sems.at[b])

    def wait_gather(p, l, b):
        pltpu.make_async_copy(relu_hbm.at[c].at[src_buf.at[p, l]], rows.at[b],
                              g_sems.at[b]).wait()

    def start_scatter(p, l, b):
        pltpu.async_copy(rows.at[b], acc.at[dst_buf.at[p, l]],
                         s_sems.at[b], add=True)

    def wait_scatter(p, l, b):
        pltpu.make_async_copy(rows.at[b], acc.at[dst_buf.at[p, l]],
                              s_sems.at[b]).wait()

    def fire_cnt(p, l, jj):
        take = ((c == 0) & (jj < CNT_SPLIT)) | ((c == 1) & (jj >= CNT_SPLIT))

        @pl.when(take)
        def _():
            pltpu.async_copy(ones_v, cnt.at[dst_buf.at[p, l]],
                             cnt_sem, add=True)

    def step(ig, p, l, b, bg, do_gather, wait_sc):
        if do_gather:
            if wait_sc:
                wait_scatter(p, l + LAG - NBUF, bg)
            start_gather(p, l + LAG, bg)
        wait_gather(p, l, b)
        start_scatter(p, l, b)
        fire_cnt(p, l, ig * IG + l)

    pltpu.sync_copy(src_hbm.at[pl.ds(crow0, IG)], src_buf.at[0])
    pltpu.sync_copy(dst_hbm.at[pl.ds(crow0, IG)], dst_buf.at[0])

    for ig in range(N_IG):
        p = ig % 2
        if ig + 1 < N_IG:
            start_idx_load(ig + 1, 1 - p)
        for l in range(LAG):
            start_gather(p, l, l % NBUF)
        for l in range(LAG):
            step(ig, p, l, l % NBUF, (l + LAG) % NBUF,
                 do_gather=True, wait_sc=(l + LAG >= NBUF))

        @pl.loop(0, (IG - LAG - NBUF) // NBUF)
        def _(g):
            l0 = LAG + g * NBUF
            for b in range(NBUF):
                l = l0 + b
                step(ig, p, l, (b + LAG) % NBUF, (b + 2 * LAG) % NBUF,
                     do_gather=True, wait_sc=True)

        for l in range(LAG + ((IG - LAG - NBUF) // NBUF) * NBUF, IG):
            step(ig, p, l, l % NBUF, (l + LAG) % NBUF,
                 do_gather=(l + LAG < IG), wait_sc=(l + LAG >= NBUF))

        for l in range(IG - NBUF, IG):
            wait_scatter(p, l, l % NBUF)
        if ig + 1 < N_IG:
            wait_idx_load(1 - p)

    @pl.loop(0, CNT_SPLIT)
    def _(i):
        pltpu.make_async_copy(ones_v, cnt.at[dst_buf.at[0, 0]],
                              cnt_sem).wait()

    plsc.subcore_barrier()

    for k in range(ROWS_PER_TILE // ZROWS):
        r = row0 + k * ZROWS
        pltpu.sync_copy(acc.at[pl.ds(r, ZROWS)], part_out.at[c, pl.ds(r, ZROWS)])
        pltpu.sync_copy(cnt.at[pl.ds(r, ZROWS)], cnt_out.at[c, pl.ds(r, ZROWS)])


_agg_sc = functools.partial(
    pl.kernel,
    out_type=(
        jax.ShapeDtypeStruct((NC, N_PAD, DH), jnp.float32),
        jax.ShapeDtypeStruct((NC, N_PAD, CW), jnp.float32),
    ),
    mesh=plsc.VectorSubcoreMesh(core_axis_name="c", subcore_axis_name="s"),
    compiler_params=pltpu.CompilerParams(use_tc_tiling_on_sc=False),
    scratch_types=[
        pltpu.VMEM_SHARED((N_PAD, DH), jnp.float32),
        pltpu.VMEM_SHARED((N_PAD, CW), jnp.float32),
        pltpu.VMEM((2, IG, CHUNK), jnp.int32),
        pltpu.VMEM((2, IG, CHUNK), jnp.int32),
        pltpu.VMEM((NBUF, CHUNK, DH), jnp.float32),
        pltpu.VMEM((CHUNK, CW), jnp.float32),
        pltpu.VMEM((ZROWS, DH), jnp.float32),
        pltpu.VMEM((ZROWS, CW), jnp.float32),
        pltpu.SemaphoreType.DMA((NBUF,)),
        pltpu.SemaphoreType.DMA((NBUF,)),
        pltpu.SemaphoreType.DMA,
        pltpu.SemaphoreType.DMA,
        pltpu.SemaphoreType.DMA,
    ],
)(_agg_sc_body)


def _combine_body(part_ref, cnt_ref, x_ref, w_ref, o_ref):
    csum = cnt_ref[0, :, 0:1] + cnt_ref[1, :, 0:1]
    inv = 1.0 / jnp.maximum(csum, 1.0)
    a0 = part_ref[0] * inv + x_ref[:, 0:DH]
    a1 = part_ref[1] * inv + x_ref[:, DH:D]
    t = (jnp.dot(a0, w_ref[0:DH, :], preferred_element_type=jnp.float32)
         + jnp.dot(a1, w_ref[DH:D, :], preferred_element_type=jnp.float32))
    o_ref[...] = jnp.maximum(t, 0.0)


def _combine_tc(part, cnt, x, W):
    blk = 2000
    return pl.pallas_call(
        _combine_body,
        grid=(N_NODES // blk,),
        in_specs=[
            pl.BlockSpec((NC, blk, DH), lambda i: (0, i, 0)),
            pl.BlockSpec((NC, blk, CW), lambda i: (0, i, 0)),
            pl.BlockSpec((blk, D), lambda i: (i, 0)),
            pl.BlockSpec((D, D), lambda i: (0, 0)),
        ],
        out_specs=pl.BlockSpec((blk, D), lambda i: (i, 0)),
        out_shape=jax.ShapeDtypeStruct((N_NODES, D), jnp.float32),
    )(part, cnt, x, W)


def kernel(x, edge_index, W):
    src = edge_index[0].reshape(-1, CHUNK)
    dst = edge_index[1].reshape(-1, CHUNK)
    relu2 = _relu_tc(x)
    part, cnt = _agg_sc(relu2, src, dst)
    return _combine_tc(part, cnt, x, W)


# speedup vs baseline: 12.6760x; 1.0559x over previous
import functools

import jax
import jax.numpy as jnp
from jax import lax
from jax.experimental import pallas as pl
from jax.experimental.pallas import tpu as pltpu
from jax.experimental.pallas import tpu_sc as plsc

N_NODES = 10000
N_EDGES = 320000
D = 128
DH = D // 2

NC = 2
NS = 16
E_PER_TILE = N_EDGES // NS
CHUNK = 80
N_CHUNKS = E_PER_TILE // CHUNK
NBUF = 4
LAG = 2
IG = 50
N_IG = N_CHUNKS // IG
N_PAD = 10240
ROWS_PER_TILE = N_PAD // NS
ZROWS = 64
CW = 16
CNT_SPLIT = N_CHUNKS // 2


def _relu_body(x_ref, o_ref):
    o_ref[0] = jnp.maximum(x_ref[:, 0:DH], 0.0)
    o_ref[1] = jnp.maximum(x_ref[:, DH:D], 0.0)


def _relu_tc(x):
    blk = 2000
    return pl.pallas_call(
        _relu_body,
        grid=(N_NODES // blk,),
        in_specs=[pl.BlockSpec((blk, D), lambda i: (i, 0))],
        out_specs=pl.BlockSpec((NC, blk, DH), lambda i: (0, i, 0)),
        out_shape=jax.ShapeDtypeStruct((NC, N_NODES, DH), jnp.float32),
    )(x)


def _agg_sc_body(relu_hbm, edge_hbm, part_out, cnt_out,
                 acc, cnt, src_buf, dst_buf, rows, ones_v, zf, zc,
                 g_sems, s_sems, cnt_sem, is_sem, id_sem):
    c = lax.axis_index("c")
    s = lax.axis_index("s")

    @pl.loop(0, ZROWS)
    def _(i):
        for j in range(DH // 16):
            zf[i, pl.ds(j * 16, 16)] = jnp.zeros((16,), jnp.float32)
        zc[i, :] = jnp.zeros((16,), jnp.float32)

    @pl.loop(0, CHUNK)
    def _(i):
        ones_v[i, :] = jnp.ones((16,), jnp.float32)

    row0 = s * ROWS_PER_TILE
    for k in range(ROWS_PER_TILE // ZROWS):
        pltpu.sync_copy(zf, acc.at[pl.ds(row0 + k * ZROWS, ZROWS)])
        pltpu.sync_copy(zc, cnt.at[pl.ds(row0 + k * ZROWS, ZROWS)])
    plsc.subcore_barrier()

    crow0 = s * N_CHUNKS

    def start_idx_load(ig, p):
        r = crow0 + ig * IG
        pltpu.async_copy(edge_hbm.at[0, pl.ds(r, IG)], src_buf.at[p], is_sem)
        pltpu.async_copy(edge_hbm.at[1, pl.ds(r, IG)], dst_buf.at[p], id_sem)

    def wait_idx_load(p):
        pltpu.make_async_copy(edge_hbm.at[0, pl.ds(crow0, IG)], src_buf.at[p],
                              is_sem).wait()
        pltpu.make_async_copy(edge_hbm.at[1, pl.ds(crow0, IG)], dst_buf.at[p],
                              id_sem).wait()

    def start_gather(p, l, b):
        pltpu.async_copy(relu_hbm.at[c].at[src_buf.at[p, l]], rows.at[b],
                         g_sems.at[b])

    def wait_gather(p, l, b):
        pltpu.make_async_copy(relu_hbm.at[c].at[src_buf.at[p, l]], rows.at[b],
                              g_sems.at[b]).wait()

    def start_scatter(p, l, b):
        pltpu.async_copy(rows.at[b], acc.at[dst_buf.at[p, l]],
                         s_sems.at[b], add=True)

    def wait_scatter(p, l, b):
        pltpu.make_async_copy(rows.at[b], acc.at[dst_buf.at[p, l]],
                              s_sems.at[b]).wait()

    def fire_cnt(p, l, jj):
        take = ((c == 0) & (jj < CNT_SPLIT)) | ((c == 1) & (jj >= CNT_SPLIT))

        @pl.when(take)
        def _():
            pltpu.async_copy(ones_v, cnt.at[dst_buf.at[p, l]],
                             cnt_sem, add=True)

    def step(ig, p, l, b, bg, do_gather, wait_sc):
        if do_gather:
            if wait_sc:
                wait_scatter(p, l + LAG - NBUF, bg)
            start_gather(p, l + LAG, bg)
        wait_gather(p, l, b)
        start_scatter(p, l, b)
        fire_cnt(p, l, ig * IG + l)

    pltpu.sync_copy(edge_hbm.at[0, pl.ds(crow0, IG)], src_buf.at[0])
    pltpu.sync_copy(edge_hbm.at[1, pl.ds(crow0, IG)], dst_buf.at[0])

    for ig in range(N_IG):
        p = ig % 2
        if ig + 1 < N_IG:
            start_idx_load(ig + 1, 1 - p)
        for l in range(LAG):
            start_gather(p, l, l % NBUF)
        for l in range(LAG):
            step(ig, p, l, l % NBUF, (l + LAG) % NBUF,
                 do_gather=True, wait_sc=(l + LAG >= NBUF))

        @pl.loop(0, (IG - LAG - NBUF) // NBUF)
        def _(g):
            l0 = LAG + g * NBUF
            for b in range(NBUF):
                l = l0 + b
                step(ig, p, l, (b + LAG) % NBUF, (b + 2 * LAG) % NBUF,
                     do_gather=True, wait_sc=True)

        for l in range(LAG + ((IG - LAG - NBUF) // NBUF) * NBUF, IG):
            step(ig, p, l, l % NBUF, (l + LAG) % NBUF,
                 do_gather=(l + LAG < IG), wait_sc=(l + LAG >= NBUF))

        for l in range(IG - NBUF, IG):
            wait_scatter(p, l, l % NBUF)
        if ig + 1 < N_IG:
            wait_idx_load(1 - p)

    @pl.loop(0, CNT_SPLIT)
    def _(i):
        pltpu.make_async_copy(ones_v, cnt.at[dst_buf.at[0, 0]],
                              cnt_sem).wait()

    plsc.subcore_barrier()

    for k in range(ROWS_PER_TILE // ZROWS):
        r = row0 + k * ZROWS
        pltpu.sync_copy(acc.at[pl.ds(r, ZROWS)], part_out.at[c, pl.ds(r, ZROWS)])
        pltpu.sync_copy(cnt.at[pl.ds(r, ZROWS)], cnt_out.at[c, pl.ds(r, ZROWS)])


_agg_sc = functools.partial(
    pl.kernel,
    out_type=(
        jax.ShapeDtypeStruct((NC, N_PAD, DH), jnp.float32),
        jax.ShapeDtypeStruct((NC, N_PAD, CW), jnp.float32),
    ),
    mesh=plsc.VectorSubcoreMesh(core_axis_name="c", subcore_axis_name="s"),
    compiler_params=pltpu.CompilerParams(use_tc_tiling_on_sc=False),
    scratch_types=[
        pltpu.VMEM_SHARED((N_PAD, DH), jnp.float32),
        pltpu.VMEM_SHARED((N_PAD, CW), jnp.float32),
        pltpu.VMEM((2, IG, CHUNK), jnp.int32),
        pltpu.VMEM((2, IG, CHUNK), jnp.int32),
        pltpu.VMEM((NBUF, CHUNK, DH), jnp.float32),
        pltpu.VMEM((CHUNK, CW), jnp.float32),
        pltpu.VMEM((ZROWS, DH), jnp.float32),
        pltpu.VMEM((ZROWS, CW), jnp.float32),
        pltpu.SemaphoreType.DMA((NBUF,)),
        pltpu.SemaphoreType.DMA((NBUF,)),
        pltpu.SemaphoreType.DMA,
        pltpu.SemaphoreType.DMA,
        pltpu.SemaphoreType.DMA,
    ],
)(_agg_sc_body)


def _combine_body(part_ref, cnt_ref, x_ref, w_ref, o_ref):
    csum = cnt_ref[0, :, 0:1] + cnt_ref[1, :, 0:1]
    inv = 1.0 / jnp.maximum(csum, 1.0)
    a0 = part_ref[0] * inv + x_ref[:, 0:DH]
    a1 = part_ref[1] * inv + x_ref[:, DH:D]
    t = (jnp.dot(a0, w_ref[0:DH, :], preferred_element_type=jnp.float32)
         + jnp.dot(a1, w_ref[DH:D, :], preferred_element_type=jnp.float32))
    o_ref[...] = jnp.maximum(t, 0.0)


def _combine_tc(part, cnt, x, W):
    blk = 2000
    return pl.pallas_call(
        _combine_body,
        grid=(N_NODES // blk,),
        in_specs=[
            pl.BlockSpec((NC, blk, DH), lambda i: (0, i, 0)),
            pl.BlockSpec((NC, blk, CW), lambda i: (0, i, 0)),
            pl.BlockSpec((blk, D), lambda i: (i, 0)),
            pl.BlockSpec((D, D), lambda i: (0, 0)),
        ],
        out_specs=pl.BlockSpec((blk, D), lambda i: (i, 0)),
        out_shape=jax.ShapeDtypeStruct((N_NODES, D), jnp.float32),
    )(part, cnt, x, W)


def kernel(x, edge_index, W):
    edge3 = edge_index.reshape(2, N_EDGES // CHUNK, CHUNK)
    relu2 = _relu_tc(x)
    part, cnt = _agg_sc(relu2, edge3)
    return _combine_tc(part, cnt, x, W)


# speedup vs baseline: 13.3912x; 1.0564x over previous
import functools

import jax
import jax.numpy as jnp
from jax import lax
from jax.experimental import pallas as pl
from jax.experimental.pallas import tpu as pltpu
from jax.experimental.pallas import tpu_sc as plsc

N_NODES = 10000
N_EDGES = 320000
D = 128
DH = D // 2

NC = 2
NS = 16
E_PER_TILE = N_EDGES // NS
CHUNK = 80
N_CHUNKS = E_PER_TILE // CHUNK
NBUF = 4
LAG = 2
IG = 50
N_IG = N_CHUNKS // IG
N_PAD = 10240
ROWS_PER_TILE = N_PAD // NS
ZROWS = 64
CW = 16
CNT_SPLIT = N_CHUNKS // 2


def _relu_body(x_ref, o_ref):
    o_ref[0] = jnp.maximum(x_ref[:, 0:DH], 0.0)
    o_ref[1] = jnp.maximum(x_ref[:, DH:D], 0.0)


def _relu_tc(x):
    blk = 2000
    return pl.pallas_call(
        _relu_body,
        grid=(N_NODES // blk,),
        in_specs=[pl.BlockSpec((blk, D), lambda i: (i, 0))],
        out_specs=pl.BlockSpec((NC, blk, DH), lambda i: (0, i, 0)),
        out_shape=jax.ShapeDtypeStruct((NC, N_NODES, DH), jnp.float32),
    )(x)


def _agg_sc_body(relu_hbm, edge_hbm, part_out, cnt_out,
                 acc, cnt, src_buf, dst_buf, rows, ones_v, zf, zc,
                 g_sems, s_sems, cnt_sem, is_sem, id_sem):
    c = lax.axis_index("c")
    s = lax.axis_index("s")

    @pl.loop(0, ZROWS)
    def _(i):
        for j in range(DH // 16):
            zf[i, pl.ds(j * 16, 16)] = jnp.zeros((16,), jnp.float32)
        zc[i, :] = jnp.zeros((16,), jnp.float32)

    @pl.loop(0, CHUNK)
    def _(i):
        ones_v[i, :] = jnp.ones((16,), jnp.float32)

    row0 = s * ROWS_PER_TILE
    for k in range(ROWS_PER_TILE // ZROWS):
        pltpu.sync_copy(zf, acc.at[pl.ds(row0 + k * ZROWS, ZROWS)])
        pltpu.sync_copy(zc, cnt.at[pl.ds(row0 + k * ZROWS, ZROWS)])
    plsc.subcore_barrier()

    crow0 = s * N_CHUNKS

    def start_idx_load(ig, p):
        r = crow0 + ig * IG
        pltpu.async_copy(edge_hbm.at[0, pl.ds(r, IG)], src_buf.at[p], is_sem)
        pltpu.async_copy(edge_hbm.at[1, pl.ds(r, IG)], dst_buf.at[p], id_sem)

    def wait_idx_load(p):
        pltpu.make_async_copy(edge_hbm.at[0, pl.ds(crow0, IG)], src_buf.at[p],
                              is_sem).wait()
        pltpu.make_async_copy(edge_hbm.at[1, pl.ds(crow0, IG)], dst_buf.at[p],
                              id_sem).wait()

    def start_gather(p, l, b):
        pltpu.async_copy(relu_hbm.at[c].at[src_buf.at[p, l]], rows.at[b],
                         g_sems.at[b])

    def wait_gather(p, l, b):
        pltpu.make_async_copy(relu_hbm.at[c].at[src_buf.at[p, l]], rows.at[b],
                              g_sems.at[b]).wait()

    def start_scatter(p, l, b):
        pltpu.async_copy(rows.at[b], acc.at[dst_buf.at[p, l]],
                         s_sems.at[b], add=True)

    def wait_scatter(p, l, b):
        pltpu.make_async_copy(rows.at[b], acc.at[dst_buf.at[p, l]],
                              s_sems.at[b]).wait()

    def fire_cnt(p, l, jj):
        take = ((c == 0) & (jj < CNT_SPLIT)) | ((c == 1) & (jj >= CNT_SPLIT))

        @pl.when(take)
        def _():
            pltpu.async_copy(ones_v, cnt.at[dst_buf.at[p, l]],
                             cnt_sem, add=True)

    def step(ig, p, l, b, bg, do_gather, wait_sc):
        if do_gather:
            if wait_sc:
                wait_scatter(p, l + LAG - NBUF, bg)
            start_gather(p, l + LAG, bg)
        wait_gather(p, l, b)
        start_scatter(p, l, b)
        fire_cnt(p, l, ig * IG + l)

    pltpu.sync_copy(edge_hbm.at[0, pl.ds(crow0, IG)], src_buf.at[0])
    pltpu.sync_copy(edge_hbm.at[1, pl.ds(crow0, IG)], dst_buf.at[0])

    for ig in range(N_IG):
        p = ig % 2
        if ig + 1 < N_IG:
            start_idx_load(ig + 1, 1 - p)
        for l in range(LAG):
            start_gather(p, l, l % NBUF)
        for l in range(LAG):
            step(ig, p, l, l % NBUF, (l + LAG) % NBUF,
                 do_gather=True, wait_sc=(l + LAG >= NBUF))

        @pl.loop(0, (IG - LAG - NBUF) // NBUF)
        def _(g):
            l0 = LAG + g * NBUF
            for b in range(NBUF):
                l = l0 + b
                step(ig, p, l, (b + LAG) % NBUF, (b + 2 * LAG) % NBUF,
                     do_gather=True, wait_sc=True)

        for l in range(LAG + ((IG - LAG - NBUF) // NBUF) * NBUF, IG):
            step(ig, p, l, l % NBUF, (l + LAG) % NBUF,
                 do_gather=(l + LAG < IG), wait_sc=(l + LAG >= NBUF))

        for l in range(IG - NBUF, IG):
            wait_scatter(p, l, l % NBUF)
        if ig + 1 < N_IG:
            wait_idx_load(1 - p)

    @pl.loop(0, CNT_SPLIT)
    def _(i):
        pltpu.make_async_copy(ones_v, cnt.at[dst_buf.at[0, 0]],
                              cnt_sem).wait()

    plsc.subcore_barrier()

    for k in range(ROWS_PER_TILE // ZROWS):
        r = row0 + k * ZROWS
        pltpu.sync_copy(acc.at[pl.ds(r, ZROWS)],
                        part_out.at[pl.ds(r, ZROWS), pl.ds(c * DH, DH)])
        pltpu.sync_copy(cnt.at[pl.ds(r, ZROWS)], cnt_out.at[c, pl.ds(r, ZROWS)])


_agg_sc = functools.partial(
    pl.kernel,
    out_type=(
        jax.ShapeDtypeStruct((N_PAD, D), jnp.float32),
        jax.ShapeDtypeStruct((NC, N_PAD, CW), jnp.float32),
    ),
    mesh=plsc.VectorSubcoreMesh(core_axis_name="c", subcore_axis_name="s"),
    compiler_params=pltpu.CompilerParams(use_tc_tiling_on_sc=False),
    scratch_types=[
        pltpu.VMEM_SHARED((N_PAD, DH), jnp.float32),
        pltpu.VMEM_SHARED((N_PAD, CW), jnp.float32),
        pltpu.VMEM((2, IG, CHUNK), jnp.int32),
        pltpu.VMEM((2, IG, CHUNK), jnp.int32),
        pltpu.VMEM((NBUF, CHUNK, DH), jnp.float32),
        pltpu.VMEM((CHUNK, CW), jnp.float32),
        pltpu.VMEM((ZROWS, DH), jnp.float32),
        pltpu.VMEM((ZROWS, CW), jnp.float32),
        pltpu.SemaphoreType.DMA((NBUF,)),
        pltpu.SemaphoreType.DMA((NBUF,)),
        pltpu.SemaphoreType.DMA,
        pltpu.SemaphoreType.DMA,
        pltpu.SemaphoreType.DMA,
    ],
)(_agg_sc_body)


def _combine_body(part_ref, cnt_ref, x_ref, w_ref, o_ref):
    csum = cnt_ref[0, :, 0:1] + cnt_ref[1, :, 0:1]
    inv = 1.0 / jnp.maximum(csum, 1.0)
    a = part_ref[...] * inv + x_ref[...]
    t = jnp.dot(a, w_ref[...], preferred_element_type=jnp.float32)
    o_ref[...] = jnp.maximum(t, 0.0)


def _combine_tc(part, cnt, x, W):
    blk = 2000
    return pl.pallas_call(
        _combine_body,
        grid=(N_NODES // blk,),
        in_specs=[
            pl.BlockSpec((blk, D), lambda i: (i, 0)),
            pl.BlockSpec((NC, blk, CW), lambda i: (0, i, 0)),
            pl.BlockSpec((blk, D), lambda i: (i, 0)),
            pl.BlockSpec((D, D), lambda i: (0, 0)),
        ],
        out_specs=pl.BlockSpec((blk, D), lambda i: (i, 0)),
        out_shape=jax.ShapeDtypeStruct((N_NODES, D), jnp.float32),
    )(part, cnt, x, W)


def kernel(x, edge_index, W):
    edge3 = edge_index.reshape(2, N_EDGES // CHUNK, CHUNK)
    relu2 = _relu_tc(x)
    part, cnt = _agg_sc(relu2, edge3)
    return _combine_tc(part, cnt, x, W)


# speedup vs baseline: 14.2010x; 1.0605x over previous
import functools

import jax
import jax.numpy as jnp
from jax import lax
from jax.experimental import pallas as pl
from jax.experimental.pallas import tpu as pltpu
from jax.experimental.pallas import tpu_sc as plsc

N_NODES = 10000
N_EDGES = 320000
D = 128
DH = D // 2

NC = 2
NS = 16
E_PER_TILE = N_EDGES // NS
CHUNK = 80
N_CHUNKS = E_PER_TILE // CHUNK
NBUF = 4
LAG = 2
IG = 50
N_IG = N_CHUNKS // IG
N_PAD = 10240
ROWS_PER_TILE = N_PAD // NS
ZROWS = 64
CW = 16
CNT_SPLIT = N_CHUNKS // 2


def _relu_body(x_ref, o_ref):
    o_ref[...] = jnp.maximum(x_ref[...], 0.0)


def _relu_tc(x):
    blk = 2000
    return pl.pallas_call(
        _relu_body,
        grid=(N_NODES // blk,),
        in_specs=[pl.BlockSpec((blk, D), lambda i: (i, 0))],
        out_specs=pl.BlockSpec((blk, D), lambda i: (i, 0)),
        out_shape=jax.ShapeDtypeStruct((N_NODES, D), jnp.float32),
    )(x)


def _agg_sc_body(relu_hbm, edge_hbm, part_out, cnt_out,
                 acc, cnt, src_buf, dst_buf, rows, ones_v, zf, zc,
                 g_sems, s_sems, cnt_sem, is_sem, id_sem):
    c = lax.axis_index("c")
    s = lax.axis_index("s")

    @pl.loop(0, ZROWS)
    def _(i):
        for j in range(DH // 16):
            zf[i, pl.ds(j * 16, 16)] = jnp.zeros((16,), jnp.float32)
        zc[i, :] = jnp.zeros((16,), jnp.float32)

    @pl.loop(0, CHUNK)
    def _(i):
        ones_v[i, :] = jnp.ones((16,), jnp.float32)

    row0 = s * ROWS_PER_TILE
    for k in range(ROWS_PER_TILE // ZROWS):
        pltpu.sync_copy(zf, acc.at[pl.ds(row0 + k * ZROWS, ZROWS)])
        pltpu.sync_copy(zc, cnt.at[pl.ds(row0 + k * ZROWS, ZROWS)])
    plsc.subcore_barrier()

    crow0 = s * N_CHUNKS

    def start_idx_load(ig, p):
        r = crow0 + ig * IG
        pltpu.async_copy(edge_hbm.at[0, pl.ds(r, IG)], src_buf.at[p], is_sem)
        pltpu.async_copy(edge_hbm.at[1, pl.ds(r, IG)], dst_buf.at[p], id_sem)

    def wait_idx_load(p):
        pltpu.make_async_copy(edge_hbm.at[0, pl.ds(crow0, IG)], src_buf.at[p],
                              is_sem).wait()
        pltpu.make_async_copy(edge_hbm.at[1, pl.ds(crow0, IG)], dst_buf.at[p],
                              id_sem).wait()

    def start_gather(p, l, b):
        for k in range(CHUNK // 16):
            v = src_buf[p, l, pl.ds(k * 16, 16)]
            src_buf[p, l, pl.ds(k * 16, 16)] = v + v + c
        pltpu.async_copy(relu_hbm.at[src_buf.at[p, l]], rows.at[b],
                         g_sems.at[b])

    def wait_gather(p, l, b):
        pltpu.make_async_copy(relu_hbm.at[src_buf.at[p, l]], rows.at[b],
                              g_sems.at[b]).wait()

    def start_scatter(p, l, b):
        pltpu.async_copy(rows.at[b], acc.at[dst_buf.at[p, l]],
                         s_sems.at[b], add=True)

    def wait_scatter(p, l, b):
        pltpu.make_async_copy(rows.at[b], acc.at[dst_buf.at[p, l]],
                              s_sems.at[b]).wait()

    def fire_cnt(p, l, jj):
        take = ((c == 0) & (jj < CNT_SPLIT)) | ((c == 1) & (jj >= CNT_SPLIT))

        @pl.when(take)
        def _():
            pltpu.async_copy(ones_v, cnt.at[dst_buf.at[p, l]],
                             cnt_sem, add=True)

    def step(ig, p, l, b, bg, do_gather, wait_sc):
        if do_gather:
            if wait_sc:
                wait_scatter(p, l + LAG - NBUF, bg)
            start_gather(p, l + LAG, bg)
        wait_gather(p, l, b)
        start_scatter(p, l, b)
        fire_cnt(p, l, ig * IG + l)

    pltpu.sync_copy(edge_hbm.at[0, pl.ds(crow0, IG)], src_buf.at[0])
    pltpu.sync_copy(edge_hbm.at[1, pl.ds(crow0, IG)], dst_buf.at[0])

    for ig in range(N_IG):
        p = ig % 2
        if ig + 1 < N_IG:
            start_idx_load(ig + 1, 1 - p)
        for l in range(LAG):
            start_gather(p, l, l % NBUF)
        for l in range(LAG):
            step(ig, p, l, l % NBUF, (l + LAG) % NBUF,
                 do_gather=True, wait_sc=(l + LAG >= NBUF))

        @pl.loop(0, (IG - LAG - NBUF) // NBUF)
        def _(g):
            l0 = LAG + g * NBUF
            for b in range(NBUF):
                l = l0 + b
                step(ig, p, l, (b + LAG) % NBUF, (b + 2 * LAG) % NBUF,
                     do_gather=True, wait_sc=True)

        for l in range(LAG + ((IG - LAG - NBUF) // NBUF) * NBUF, IG):
            step(ig, p, l, l % NBUF, (l + LAG) % NBUF,
                 do_gather=(l + LAG < IG), wait_sc=(l + LAG >= NBUF))

        for l in range(IG - NBUF, IG):
            wait_scatter(p, l, l % NBUF)
        if ig + 1 < N_IG:
            wait_idx_load(1 - p)

    @pl.loop(0, CNT_SPLIT)
    def _(i):
        pltpu.make_async_copy(ones_v, cnt.at[dst_buf.at[0, 0]],
                              cnt_sem).wait()

    plsc.subcore_barrier()

    for k in range(ROWS_PER_TILE // ZROWS):
        r = row0 + k * ZROWS
        pltpu.sync_copy(acc.at[pl.ds(r, ZROWS)],
                        part_out.at[pl.ds(r, ZROWS), pl.ds(c * DH, DH)])
        pltpu.sync_copy(cnt.at[pl.ds(r, ZROWS)], cnt_out.at[c, pl.ds(r, ZROWS)])


_agg_sc = functools.partial(
    pl.kernel,
    out_type=(
        jax.ShapeDtypeStruct((N_PAD, D), jnp.float32),
        jax.ShapeDtypeStruct((NC, N_PAD, CW), jnp.float32),
    ),
    mesh=plsc.VectorSubcoreMesh(core_axis_name="c", subcore_axis_name="s"),
    compiler_params=pltpu.CompilerParams(use_tc_tiling_on_sc=False),
    scratch_types=[
        pltpu.VMEM_SHARED((N_PAD, DH), jnp.float32),
        pltpu.VMEM_SHARED((N_PAD, CW), jnp.float32),
        pltpu.VMEM((2, IG, CHUNK), jnp.int32),
        pltpu.VMEM((2, IG, CHUNK), jnp.int32),
        pltpu.VMEM((NBUF, CHUNK, DH), jnp.float32),
        pltpu.VMEM((CHUNK, CW), jnp.float32),
        pltpu.VMEM((ZROWS, DH), jnp.float32),
        pltpu.VMEM((ZROWS, CW), jnp.float32),
        pltpu.SemaphoreType.DMA((NBUF,)),
        pltpu.SemaphoreType.DMA((NBUF,)),
        pltpu.SemaphoreType.DMA,
        pltpu.SemaphoreType.DMA,
        pltpu.SemaphoreType.DMA,
    ],
)(_agg_sc_body)


def _combine_body(part_ref, cnt_ref, x_ref, w_ref, o_ref):
    csum = cnt_ref[0, :, 0:1] + cnt_ref[1, :, 0:1]
    inv = 1.0 / jnp.maximum(csum, 1.0)
    a = part_ref[...] * inv + x_ref[...]
    t = jnp.dot(a, w_ref[...], preferred_element_type=jnp.float32)
    o_ref[...] = jnp.maximum(t, 0.0)


def _combine_tc(part, cnt, x, W):
    blk = 2000
    return pl.pallas_call(
        _combine_body,
        grid=(N_NODES // blk,),
        in_specs=[
            pl.BlockSpec((blk, D), lambda i: (i, 0)),
            pl.BlockSpec((NC, blk, CW), lambda i: (0, i, 0)),
            pl.BlockSpec((blk, D), lambda i: (i, 0)),
            pl.BlockSpec((D, D), lambda i: (0, 0)),
        ],
        out_specs=pl.BlockSpec((blk, D), lambda i: (i, 0)),
        out_shape=jax.ShapeDtypeStruct((N_NODES, D), jnp.float32),
    )(part, cnt, x, W)


def kernel(x, edge_index, W):
    edge3 = edge_index.reshape(2, N_EDGES // CHUNK, CHUNK)
    relu2 = _relu_tc(x).reshape(NC * N_NODES, DH)
    part, cnt = _agg_sc(relu2, edge3)
    return _combine_tc(part, cnt, x, W)


# speedup vs baseline: 15.4301x; 1.0866x over previous
import functools

import jax
import jax.numpy as jnp
from jax import lax
from jax.experimental import pallas as pl
from jax.experimental.pallas import tpu as pltpu
from jax.experimental.pallas import tpu_sc as plsc

N_NODES = 10000
N_EDGES = 320000
D = 128
DH = D // 2

NC = 2
NS = 16
E_PER_TILE = N_EDGES // NS
CHUNK = 80
N_CHUNKS = E_PER_TILE // CHUNK
NBUF = 4
LAG = 2
IG = 50
N_IG = N_CHUNKS // IG
N_PAD = 10240
ROWS_PER_TILE = N_PAD // NS
ZROWS = 64
MROWS = 64


def _relu_body(x_ref, o_ref):
    o_ref[...] = jnp.maximum(x_ref[...], 0.0)


def _relu_tc(x):
    blk = 2000
    return pl.pallas_call(
        _relu_body,
        grid=(N_NODES // blk,),
        in_specs=[pl.BlockSpec((blk, D), lambda i: (i, 0))],
        out_specs=pl.BlockSpec((blk, D), lambda i: (i, 0)),
        out_shape=jax.ShapeDtypeStruct((N_NODES, D), jnp.float32),
    )(x)


def _agg_sc_body(relu_hbm, edge_hbm, part_out,
                 acc, cnt_st, src_buf, dst_buf, rows, hist, csum, mbuf, zf,
                 g_sems, s_sems, is_sem, id_sem):
    c = lax.axis_index("c")
    s = lax.axis_index("s")

    @pl.loop(0, ZROWS)
    def _(i):
        for j in range(DH // 16):
            zf[i, pl.ds(j * 16, 16)] = jnp.zeros((16,), jnp.float32)

    @pl.loop(0, N_PAD // 128)
    def _(i):
        for j in range(8):
            hist[pl.ds(i * 128 + j * 16, 16)] = jnp.zeros((16,), jnp.float32)

    row0 = s * ROWS_PER_TILE
    for k in range(ROWS_PER_TILE // ZROWS):
        pltpu.sync_copy(zf, acc.at[pl.ds(row0 + k * ZROWS, ZROWS)])
    plsc.subcore_barrier()

    crow0 = s * N_CHUNKS

    def start_idx_load(ig, p):
        r = crow0 + ig * IG
        pltpu.async_copy(edge_hbm.at[0, pl.ds(r, IG)], src_buf.at[p], is_sem)
        pltpu.async_copy(edge_hbm.at[1, pl.ds(r, IG)], dst_buf.at[p], id_sem)

    def wait_idx_load(p):
        pltpu.make_async_copy(edge_hbm.at[0, pl.ds(crow0, IG)], src_buf.at[p],
                              is_sem).wait()
        pltpu.make_async_copy(edge_hbm.at[1, pl.ds(crow0, IG)], dst_buf.at[p],
                              id_sem).wait()

    def start_gather(p, l, b):
        for k in range(CHUNK // 16):
            v = src_buf[p, l, pl.ds(k * 16, 16)]
            src_buf[p, l, pl.ds(k * 16, 16)] = v + v + c
        pltpu.async_copy(relu_hbm.at[src_buf.at[p, l]], rows.at[b],
                         g_sems.at[b])

    def wait_gather(p, l, b):
        pltpu.make_async_copy(relu_hbm.at[src_buf.at[p, l]], rows.at[b],
                              g_sems.at[b]).wait()

    def start_scatter(p, l, b):
        pltpu.async_copy(rows.at[b], acc.at[dst_buf.at[p, l]],
                         s_sems.at[b], add=True)

    def wait_scatter(p, l, b):
        pltpu.make_async_copy(rows.at[b], acc.at[dst_buf.at[p, l]],
                              s_sems.at[b]).wait()

    ones16 = jnp.ones((16,), jnp.float32)

    def count_chunk(p, l):
        for k in range(CHUNK // 16):
            idxv = dst_buf[p, l, pl.ds(k * 16, 16)]
            plsc.addupdate_scatter(hist, [idxv], ones16)

    def step(ig, p, l, b, bg, do_gather, wait_sc):
        if do_gather:
            if wait_sc:
                wait_scatter(p, l + LAG - NBUF, bg)
            start_gather(p, l + LAG, bg)
        wait_gather(p, l, b)
        start_scatter(p, l, b)
        count_chunk(p, l)

    pltpu.sync_copy(edge_hbm.at[0, pl.ds(crow0, IG)], src_buf.at[0])
    pltpu.sync_copy(edge_hbm.at[1, pl.ds(crow0, IG)], dst_buf.at[0])

    for ig in range(N_IG):
        p = ig % 2
        if ig + 1 < N_IG:
            start_idx_load(ig + 1, 1 - p)
        for l in range(LAG):
            start_gather(p, l, l % NBUF)
        for l in range(LAG):
            step(ig, p, l, l % NBUF, (l + LAG) % NBUF,
                 do_gather=True, wait_sc=(l + LAG >= NBUF))

        @pl.loop(0, (IG - LAG - NBUF) // NBUF)
        def _(g):
            l0 = LAG + g * NBUF
            for b in range(NBUF):
                l = l0 + b
                step(ig, p, l, (b + LAG) % NBUF, (b + 2 * LAG) % NBUF,
                     do_gather=True, wait_sc=True)

        for l in range(LAG + ((IG - LAG - NBUF) // NBUF) * NBUF, IG):
            step(ig, p, l, l % NBUF, (l + LAG) % NBUF,
                 do_gather=(l + LAG < IG), wait_sc=(l + LAG >= NBUF))

        for l in range(IG - NBUF, IG):
            wait_scatter(p, l, l % NBUF)
        if ig + 1 < N_IG:
            wait_idx_load(1 - p)

    pltpu.sync_copy(hist, cnt_st.at[s])
    plsc.subcore_barrier()

    for k in range(ROWS_PER_TILE // MROWS):
        r = row0 + k * MROWS
        pltpu.sync_copy(cnt_st.at[:, pl.ds(r, MROWS)], mbuf)
        for j in range(MROWS // 16):
            tot = mbuf[0, pl.ds(j * 16, 16)]
            for t in range(1, NS):
                tot = tot + mbuf[t, pl.ds(j * 16, 16)]
            csum[pl.ds(k * MROWS + j * 16, 16)] = tot

    for k in range(ROWS_PER_TILE // ZROWS):
        r = row0 + k * ZROWS
        pltpu.sync_copy(acc.at[pl.ds(r, ZROWS)], zf)

        @pl.loop(0, ZROWS // 16)
        def _(i16):
            cv = csum[pl.ds(k * ZROWS + i16 * 16, 16)]
            invv = 1.0 / jnp.maximum(cv, 1.0)
            for ii in range(16):
                inv = invv[ii]
                row = i16 * 16 + ii
                for j in range(DH // 16):
                    zf[row, pl.ds(j * 16, 16)] = (
                        zf[row, pl.ds(j * 16, 16)] * inv)

        pltpu.sync_copy(zf, part_out.at[pl.ds(r, ZROWS), pl.ds(c * DH, DH)])


_agg_sc = functools.partial(
    pl.kernel,
    out_type=jax.ShapeDtypeStruct((N_PAD, D), jnp.float32),
    mesh=plsc.VectorSubcoreMesh(core_axis_name="c", subcore_axis_name="s"),
    compiler_params=pltpu.CompilerParams(use_tc_tiling_on_sc=False,
                                         needs_layout_passes=False),
    scratch_types=[
        pltpu.VMEM_SHARED((N_PAD, DH), jnp.float32),
        pltpu.VMEM_SHARED((NS, N_PAD), jnp.float32),
        pltpu.VMEM((2, IG, CHUNK), jnp.int32),
        pltpu.VMEM((2, IG, CHUNK), jnp.int32),
        pltpu.VMEM((NBUF, CHUNK, DH), jnp.float32),
        pltpu.VMEM((N_PAD,), jnp.float32),
        pltpu.VMEM((ROWS_PER_TILE,), jnp.float32),
        pltpu.VMEM((NS, MROWS), jnp.float32),
        pltpu.VMEM((ZROWS, DH), jnp.float32),
        pltpu.SemaphoreType.DMA((NBUF,)),
        pltpu.SemaphoreType.DMA((NBUF,)),
        pltpu.SemaphoreType.DMA,
        pltpu.SemaphoreType.DMA,
    ],
)(_agg_sc_body)


def _combine_body(mean_ref, x_ref, w_ref, o_ref):
    a = mean_ref[...] + x_ref[...]
    t = jnp.dot(a, w_ref[...], preferred_element_type=jnp.float32)
    o_ref[...] = jnp.maximum(t, 0.0)


def _combine_tc(mean, x, W):
    blk = 2000
    return pl.pallas_call(
        _combine_body,
        grid=(N_NODES // blk,),
        in_specs=[
            pl.BlockSpec((blk, D), lambda i: (i, 0)),
            pl.BlockSpec((blk, D), lambda i: (i, 0)),
            pl.BlockSpec((D, D), lambda i: (0, 0)),
        ],
        out_specs=pl.BlockSpec((blk, D), lambda i: (i, 0)),
        out_shape=jax.ShapeDtypeStruct((N_NODES, D), jnp.float32),
    )(mean, x, W)


def kernel(x, edge_index, W):
    edge3 = edge_index.reshape(2, N_EDGES // CHUNK, CHUNK)
    relu2 = _relu_tc(x).reshape(NC * N_NODES, DH)
    mean = _agg_sc(relu2, edge3)
    return _combine_tc(mean, x, W)


# speedup vs baseline: 16.1340x; 1.0456x over previous
import functools

import jax
import jax.numpy as jnp
from jax import lax
from jax.experimental import pallas as pl
from jax.experimental.pallas import tpu as pltpu
from jax.experimental.pallas import tpu_sc as plsc

N_NODES = 10000
N_EDGES = 320000
D = 128
DH = D // 2

NC = 2
NS = 16
E_PER_TILE = N_EDGES // NS
CHUNK = 80
N_CHUNKS = E_PER_TILE // CHUNK
NBUF = 6
LAG = 3
IG = 50
N_IG = N_CHUNKS // IG
N_PAD = 10240
ROWS_PER_TILE = N_PAD // NS
ZROWS = 64
MROWS = 64


def _relu_body(x_ref, o_ref):
    o_ref[...] = jnp.maximum(x_ref[...], 0.0)


def _relu_tc(x):
    blk = 2000
    return pl.pallas_call(
        _relu_body,
        grid=(N_NODES // blk,),
        in_specs=[pl.BlockSpec((blk, D), lambda i: (i, 0))],
        out_specs=pl.BlockSpec((blk, D), lambda i: (i, 0)),
        out_shape=jax.ShapeDtypeStruct((N_NODES, D), jnp.float32),
    )(x)


def _agg_sc_body(relu_hbm, edge_hbm, part_out,
                 acc, cnt_st, src_buf, dst_buf, rows, hist, csum, mbuf, zf,
                 g_sems, s_sems, is_sem, id_sem):
    c = lax.axis_index("c")
    s = lax.axis_index("s")

    @pl.loop(0, ZROWS)
    def _(i):
        for j in range(DH // 16):
            zf[i, pl.ds(j * 16, 16)] = jnp.zeros((16,), jnp.float32)

    @pl.loop(0, N_PAD // 128)
    def _(i):
        for j in range(8):
            hist[pl.ds(i * 128 + j * 16, 16)] = jnp.zeros((16,), jnp.float32)

    row0 = s * ROWS_PER_TILE
    for k in range(ROWS_PER_TILE // ZROWS):
        pltpu.sync_copy(zf, acc.at[pl.ds(row0 + k * ZROWS, ZROWS)])
    plsc.subcore_barrier()

    crow0 = s * N_CHUNKS

    def start_idx_load(ig, p):
        r = crow0 + ig * IG
        pltpu.async_copy(edge_hbm.at[0, pl.ds(r, IG)], src_buf.at[p], is_sem)
        pltpu.async_copy(edge_hbm.at[1, pl.ds(r, IG)], dst_buf.at[p], id_sem)

    def wait_idx_load(p):
        pltpu.make_async_copy(edge_hbm.at[0, pl.ds(crow0, IG)], src_buf.at[p],
                              is_sem).wait()
        pltpu.make_async_copy(edge_hbm.at[1, pl.ds(crow0, IG)], dst_buf.at[p],
                              id_sem).wait()

    def start_gather(p, l, b):
        for k in range(CHUNK // 16):
            v = src_buf[p, l, pl.ds(k * 16, 16)]
            src_buf[p, l, pl.ds(k * 16, 16)] = v + v + c
        pltpu.async_copy(relu_hbm.at[src_buf.at[p, l]], rows.at[b],
                         g_sems.at[b])

    def wait_gather(p, l, b):
        pltpu.make_async_copy(relu_hbm.at[src_buf.at[p, l]], rows.at[b],
                              g_sems.at[b]).wait()

    def start_scatter(p, l, b):
        pltpu.async_copy(rows.at[b], acc.at[dst_buf.at[p, l]],
                         s_sems.at[b], add=True)

    def wait_scatter(p, l, b):
        pltpu.make_async_copy(rows.at[b], acc.at[dst_buf.at[p, l]],
                              s_sems.at[b]).wait()

    ones16 = jnp.ones((16,), jnp.float32)

    def count_chunk(p, l):
        for k in range(CHUNK // 16):
            idxv = dst_buf[p, l, pl.ds(k * 16, 16)]
            plsc.addupdate_scatter(hist, [idxv], ones16)

    def step(ig, p, l, b, bg, do_gather, wait_sc):
        if do_gather:
            if wait_sc:
                wait_scatter(p, l + LAG - NBUF, bg)
            start_gather(p, l + LAG, bg)
        wait_gather(p, l, b)
        start_scatter(p, l, b)
        count_chunk(p, l)

    pltpu.sync_copy(edge_hbm.at[0, pl.ds(crow0, IG)], src_buf.at[0])
    pltpu.sync_copy(edge_hbm.at[1, pl.ds(crow0, IG)], dst_buf.at[0])

    for ig in range(N_IG):
        p = ig % 2
        if ig + 1 < N_IG:
            start_idx_load(ig + 1, 1 - p)
        for l in range(LAG):
            start_gather(p, l, l % NBUF)
        for l in range(LAG):
            step(ig, p, l, l % NBUF, (l + LAG) % NBUF,
                 do_gather=True, wait_sc=(l + LAG >= NBUF))

        @pl.loop(0, (IG - LAG - NBUF) // NBUF)
        def _(g):
            l0 = LAG + g * NBUF
            for b in range(NBUF):
                l = l0 + b
                step(ig, p, l, (b + LAG) % NBUF, (b + 2 * LAG) % NBUF,
                     do_gather=True, wait_sc=True)

        for l in range(LAG + ((IG - LAG - NBUF) // NBUF) * NBUF, IG):
            step(ig, p, l, l % NBUF, (l + LAG) % NBUF,
                 do_gather=(l + LAG < IG), wait_sc=(l + LAG >= NBUF))

        for l in range(IG - NBUF, IG):
            wait_scatter(p, l, l % NBUF)
        if ig + 1 < N_IG:
            wait_idx_load(1 - p)

    pltpu.sync_copy(hist, cnt_st.at[s])
    plsc.subcore_barrier()

    for k in range(ROWS_PER_TILE // MROWS):
        r = row0 + k * MROWS
        pltpu.sync_copy(cnt_st.at[:, pl.ds(r, MROWS)], mbuf)
        for j in range(MROWS // 16):
            tot = mbuf[0, pl.ds(j * 16, 16)]
            for t in range(1, NS):
                tot = tot + mbuf[t, pl.ds(j * 16, 16)]
            csum[pl.ds(k * MROWS + j * 16, 16)] = tot

    for k in range(ROWS_PER_TILE // ZROWS):
        r = row0 + k * ZROWS
        pltpu.sync_copy(acc.at[pl.ds(r, ZROWS)], zf)

        @pl.loop(0, ZROWS // 16)
        def _(i16):
            cv = csum[pl.ds(k * ZROWS + i16 * 16, 16)]
            invv = 1.0 / jnp.maximum(cv, 1.0)
            for ii in range(16):
                inv = invv[ii]
                row = i16 * 16 + ii
                for j in range(DH // 16):
                    zf[row, pl.ds(j * 16, 16)] = (
                        zf[row, pl.ds(j * 16, 16)] * inv)

        pltpu.sync_copy(zf, part_out.at[pl.ds(r, ZROWS), pl.ds(c * DH, DH)])


_agg_sc = functools.partial(
    pl.kernel,
    out_type=jax.ShapeDtypeStruct((N_PAD, D), jnp.float32),
    mesh=plsc.VectorSubcoreMesh(core_axis_name="c", subcore_axis_name="s"),
    compiler_params=pltpu.CompilerParams(use_tc_tiling_on_sc=False,
                                         needs_layout_passes=False),
    scratch_types=[
        pltpu.VMEM_SHARED((N_PAD, DH), jnp.float32),
        pltpu.VMEM_SHARED((NS, N_PAD), jnp.float32),
        pltpu.VMEM((2, IG, CHUNK), jnp.int32),
        pltpu.VMEM((2, IG, CHUNK), jnp.int32),
        pltpu.VMEM((NBUF, CHUNK, DH), jnp.float32),
        pltpu.VMEM((N_PAD,), jnp.float32),
        pltpu.VMEM((ROWS_PER_TILE,), jnp.float32),
        pltpu.VMEM((NS, MROWS), jnp.float32),
        pltpu.VMEM((ZROWS, DH), jnp.float32),
        pltpu.SemaphoreType.DMA((NBUF,)),
        pltpu.SemaphoreType.DMA((NBUF,)),
        pltpu.SemaphoreType.DMA,
        pltpu.SemaphoreType.DMA,
    ],
)(_agg_sc_body)


def _combine_body(mean_ref, x_ref, w_ref, o_ref):
    a = mean_ref[...] + x_ref[...]
    t = jnp.dot(a, w_ref[...], preferred_element_type=jnp.float32)
    o_ref[...] = jnp.maximum(t, 0.0)


def _combine_tc(mean, x, W):
    blk = 2000
    return pl.pallas_call(
        _combine_body,
        grid=(N_NODES // blk,),
        in_specs=[
            pl.BlockSpec((blk, D), lambda i: (i, 0)),
            pl.BlockSpec((blk, D), lambda i: (i, 0)),
            pl.BlockSpec((D, D), lambda i: (0, 0)),
        ],
        out_specs=pl.BlockSpec((blk, D), lambda i: (i, 0)),
        out_shape=jax.ShapeDtypeStruct((N_NODES, D), jnp.float32),
    )(mean, x, W)


def kernel(x, edge_index, W):
    edge3 = edge_index.reshape(2, N_EDGES // CHUNK, CHUNK)
    relu2 = _relu_tc(x).reshape(NC * N_NODES, DH)
    mean = _agg_sc(relu2, edge3)
    return _combine_tc(mean, x, W)
